# Initial kernel scaffold; baseline (speedup 1.0000x reference)
import jax
import jax.numpy as jnp
from jax.experimental import pallas as pl


def kernel(x, edge_index, W1, W2):
    raise NotImplementedError("write your pallas kernel here")


# speedup vs baseline: 7.2033x; 7.2033x over previous
import functools

import jax
import jax.numpy as jnp
from jax import lax
from jax.experimental import pallas as pl
from jax.experimental.pallas import tpu as pltpu
from jax.experimental.pallas import tpu_sc as plsc

N = 10000
E = 320000
D = 128

NC = 2
NS = 16
NW = NC * NS
K = 80
CPT = E // NW // K
G = 25
NG = CPT // G
ZR = K
NZB = N // ZR
LANES = 16

ROWB = 2000


def _pre_body(x_ref, w1_ref, p_ref, q_ref):
    xb = x_ref[...]
    w = w1_ref[...]
    p_ref[...] = jnp.dot(xb, w[:D, :], preferred_element_type=jnp.float32)
    q_ref[...] = jnp.dot(xb, w[D:, :], preferred_element_type=jnp.float32)


def _post_body(a0_ref, a1_ref, w2_ref, o_ref):
    o_ref[...] = jnp.dot(a0_ref[...] + a1_ref[...], w2_ref[...],
                         preferred_element_type=jnp.float32)


_edge_mesh = plsc.VectorSubcoreMesh(
    core_axis_name="c", subcore_axis_name="s", num_cores=NC, num_subcores=NS)


@functools.partial(
    pl.kernel,
    mesh=_edge_mesh,
    out_type=jax.ShapeDtypeStruct((NC, N, D), jnp.float32),
    scratch_types=[
        pltpu.VMEM((G, K), jnp.int32),
        pltpu.VMEM((G, K), jnp.int32),
        pltpu.VMEM((K, D), jnp.float32),
        pltpu.VMEM((K, D), jnp.float32),
        pltpu.VMEM_SHARED((N, D), jnp.float32),
        pltpu.SemaphoreType.DMA,
        pltpu.SemaphoreType.DMA,
    ],
)
def _edge_kernel(p_hbm, q_hbm, src_hbm, dst_hbm, out_hbm,
                 src_v, dst_v, prow, qrow, agg, sem_p, sem_q):
    c = lax.axis_index("c")
    s = lax.axis_index("s")
    wid = s * NC + c

    @pl.loop(0, K)
    def _zero_fill(r):
        for j in range(D // LANES):
            prow[r, pl.ds(j * LANES, LANES)] = jnp.zeros((LANES,), jnp.float32)

    @pl.loop(s, NZB, step=NS)
    def _zero_agg(k):
        pltpu.sync_copy(prow, agg.at[pl.ds(k * ZR, ZR)])

    plsc.subcore_barrier()

    @pl.loop(0, NG)
    def _group(g):
        pltpu.sync_copy(src_hbm.at[wid].at[g], src_v)
        pltpu.sync_copy(dst_hbm.at[wid].at[g], dst_v)

        @pl.loop(0, G)
        def _chunk(i):
            gp = pltpu.async_copy(p_hbm.at[src_v.at[i]], prow, sem_p)
            gq = pltpu.async_copy(q_hbm.at[dst_v.at[i]], qrow, sem_q)
            gp.wait()
            gq.wait()

            @pl.loop(0, K)
            def _row(r):
                for j in range(D // LANES):
                    sl = pl.ds(j * LANES, LANES)
                    prow[r, sl] = jnp.maximum(prow[r, sl] + qrow[r, sl], 0.0)

            pltpu.sync_copy(prow, agg.at[dst_v.at[i]], add=True)

    plsc.subcore_barrier()

    @pl.loop(s, NZB, step=NS)
    def _writeback(k):
        sl = pl.ds(k * ZR, ZR)
        pltpu.sync_copy(agg.at[sl], out_hbm.at[c].at[sl])


def kernel(x, edge_index, W1, W2):
    src3d = edge_index[0].reshape(NW, NG, G, K)
    dst3d = edge_index[1].reshape(NW, NG, G, K)

    p, q = pl.pallas_call(
        _pre_body,
        grid=(N // ROWB,),
        in_specs=[
            pl.BlockSpec((ROWB, D), lambda i: (i, 0)),
            pl.BlockSpec((2 * D, D), lambda i: (0, 0)),
        ],
        out_specs=[
            pl.BlockSpec((ROWB, D), lambda i: (i, 0)),
            pl.BlockSpec((ROWB, D), lambda i: (i, 0)),
        ],
        out_shape=[
            jax.ShapeDtypeStruct((N, D), jnp.float32),
            jax.ShapeDtypeStruct((N, D), jnp.float32),
        ],
    )(x, W1)

    parts = _edge_kernel(p, q, src3d, dst3d)

    out = pl.pallas_call(
        _post_body,
        grid=(N // ROWB,),
        in_specs=[
            pl.BlockSpec((ROWB, D), lambda i: (i, 0)),
            pl.BlockSpec((ROWB, D), lambda i: (i, 0)),
            pl.BlockSpec((D, D), lambda i: (0, 0)),
        ],
        out_specs=pl.BlockSpec((ROWB, D), lambda i: (i, 0)),
        out_shape=jax.ShapeDtypeStruct((N, D), jnp.float32),
    )(parts[0], parts[1], W2)

    return out


# speedup vs baseline: 7.8364x; 1.0879x over previous
import functools

import jax
import jax.numpy as jnp
from jax import lax
from jax.experimental import pallas as pl
from jax.experimental.pallas import tpu as pltpu
from jax.experimental.pallas import tpu_sc as plsc

N = 10000
E = 320000
D = 128

NC = 2
NS = 16
NW = NC * NS
K = 40
CPT = E // NW // K
G = 50
NG = CPT // G
ZR = K
NZB = N // ZR
LANES = 16

ROWB = 2000


def _pre_body(x_ref, w1_ref, p_ref, q_ref):
    xb = x_ref[...]
    w = w1_ref[...]
    p_ref[...] = jnp.dot(xb, w[:D, :], preferred_element_type=jnp.float32)
    q_ref[...] = jnp.dot(xb, w[D:, :], preferred_element_type=jnp.float32)


def _post_body(a0_ref, a1_ref, w2_ref, o_ref):
    o_ref[...] = jnp.dot(a0_ref[...] + a1_ref[...], w2_ref[...],
                         preferred_element_type=jnp.float32)


_edge_mesh = plsc.VectorSubcoreMesh(
    core_axis_name="c", subcore_axis_name="s", num_cores=NC, num_subcores=NS)


@functools.partial(
    pl.kernel,
    mesh=_edge_mesh,
    out_type=jax.ShapeDtypeStruct((NC, N, D), jnp.float32),
    scratch_types=[
        pltpu.VMEM((G, K), jnp.int32),
        pltpu.VMEM((G, K), jnp.int32),
        pltpu.VMEM((K, D), jnp.float32),
        pltpu.VMEM((K, D), jnp.float32),
        pltpu.VMEM((K, D), jnp.float32),
        pltpu.VMEM((K, D), jnp.float32),
        pltpu.VMEM_SHARED((N, D), jnp.float32),
        pltpu.SemaphoreType.DMA,
        pltpu.SemaphoreType.DMA,
    ],
)
def _edge_kernel(p_hbm, q_hbm, src_hbm, dst_hbm, out_hbm,
                 src_v, dst_v, p0, p1, q0, q1, agg, sg0, sg1):
    c = lax.axis_index("c")
    s = lax.axis_index("s")
    wid = s * NC + c

    @pl.loop(0, K)
    def _zero_fill(r):
        for j in range(D // LANES):
            p0[r, pl.ds(j * LANES, LANES)] = jnp.zeros((LANES,), jnp.float32)

    @pl.loop(s, NZB, step=NS)
    def _zero_agg(k):
        pltpu.sync_copy(p0, agg.at[pl.ds(k * ZR, ZR)])

    plsc.subcore_barrier()

    bufs = ((p0, q0, sg0), (p1, q1, sg1))

    def _issue(i, pb, qb, sg):
        pltpu.async_copy(p_hbm.at[src_v.at[i]], pb, sg)
        pltpu.async_copy(q_hbm.at[dst_v.at[i]], qb, sg)

    def _chunk(i, b):
        pb, qb, sg = bufs[b]
        pltpu.make_async_copy(p_hbm.at[src_v.at[i]], pb, sg).wait()
        pltpu.make_async_copy(q_hbm.at[dst_v.at[i]], qb, sg).wait()

        opb, oqb, osg = bufs[1 - b]

        @pl.when(i + 1 < G)
        def _prefetch():
            _issue(i + 1, opb, oqb, osg)

        @pl.loop(0, K)
        def _row(r):
            for j in range(D // LANES):
                sl = pl.ds(j * LANES, LANES)
                pb[r, sl] = jnp.maximum(pb[r, sl] + qb[r, sl], 0.0)

        pltpu.sync_copy(pb, agg.at[dst_v.at[i]], add=True)

    @pl.loop(0, NG)
    def _group(g):
        pltpu.sync_copy(src_hbm.at[wid].at[g], src_v)
        pltpu.sync_copy(dst_hbm.at[wid].at[g], dst_v)
        _issue(0, p0, q0, sg0)

        @pl.loop(0, G, step=2)
        def _pair(i):
            _chunk(i, 0)
            _chunk(i + 1, 1)

    plsc.subcore_barrier()

    @pl.loop(s, NZB, step=NS)
    def _writeback(k):
        sl = pl.ds(k * ZR, ZR)
        pltpu.sync_copy(agg.at[sl], out_hbm.at[c].at[sl])


def kernel(x, edge_index, W1, W2):
    src3d = edge_index[0].reshape(NW, NG, G, K)
    dst3d = edge_index[1].reshape(NW, NG, G, K)

    p, q = pl.pallas_call(
        _pre_body,
        grid=(N // ROWB,),
        in_specs=[
            pl.BlockSpec((ROWB, D), lambda i: (i, 0)),
            pl.BlockSpec((2 * D, D), lambda i: (0, 0)),
        ],
        out_specs=[
            pl.BlockSpec((ROWB, D), lambda i: (i, 0)),
            pl.BlockSpec((ROWB, D), lambda i: (i, 0)),
        ],
        out_shape=[
            jax.ShapeDtypeStruct((N, D), jnp.float32),
            jax.ShapeDtypeStruct((N, D), jnp.float32),
        ],
    )(x, W1)

    parts = _edge_kernel(p, q, src3d, dst3d)

    out = pl.pallas_call(
        _post_body,
        grid=(N // ROWB,),
        in_specs=[
            pl.BlockSpec((ROWB, D), lambda i: (i, 0)),
            pl.BlockSpec((ROWB, D), lambda i: (i, 0)),
            pl.BlockSpec((D, D), lambda i: (0, 0)),
        ],
        out_specs=pl.BlockSpec((ROWB, D), lambda i: (i, 0)),
        out_shape=jax.ShapeDtypeStruct((N, D), jnp.float32),
    )(parts[0], parts[1], W2)

    return out
